# Initial kernel scaffold; baseline (speedup 1.0000x reference)
import functools

import jax
import jax.numpy as jnp
from jax import lax
from jax.experimental import pallas as pl
from jax.experimental.pallas import tpu as pltpu
from jax.experimental.pallas import tpu_sc as plsc

_LANES = 16
_IDX_CHUNK = 128


@functools.lru_cache(maxsize=None)
def _make_sc_kernel(batch, list_len, dim):
    info = plsc.get_sparse_core_info()
    num_workers = info.num_cores * info.num_subcores
    users_per_w = batch // num_workers
    assert batch % num_workers == 0
    ngroups = (list_len + _LANES - 1) // _LANES
    pad_rows = ngroups * _LANES
    sub_sizes = []
    rem = list_len
    while rem > 0:
        s = min(_IDX_CHUNK, rem)
        sub_sizes.append(s)
        rem -= s

    mesh = plsc.VectorSubcoreMesh(core_axis_name="c", subcore_axis_name="s")

    @functools.partial(
        pl.kernel,
        out_type=jax.ShapeDtypeStruct((batch * list_len,), jnp.float32),
        mesh=mesh,
        scratch_types=[
            pltpu.VMEM((users_per_w,), jnp.int32),
            pltpu.VMEM((users_per_w, dim), jnp.float32),
            pltpu.VMEM((users_per_w * list_len,), jnp.int32),
            pltpu.VMEM((pad_rows, dim), jnp.float32),
            pltpu.VMEM((pad_rows, dim), jnp.float32),
            pltpu.VMEM((pad_rows,), jnp.float32),
            pltpu.SemaphoreType.DMA,
            pltpu.SemaphoreType.DMA,
            pltpu.SemaphoreType.DMA,
        ],
    )
    def sc_kernel(uid_hbm, iid_hbm, utab_hbm, itab_hbm, out_hbm,
                  uidx, urows, iidx, rows0, rows1, scores, sem0, sem1, semu):
        rows = (rows0, rows1)
        sems = (sem0, sem1)
        wid = lax.axis_index("s") * info.num_cores + lax.axis_index("c")
        ubase = wid * users_per_w

        pltpu.sync_copy(uid_hbm.at[pl.ds(ubase, users_per_w)], uidx)
        pltpu.sync_copy(
            iid_hbm.at[pl.ds(ubase * list_len, users_per_w * list_len)], iidx)
        pltpu.async_copy(utab_hbm.at[uidx], urows, semu).wait()

        def gather_descs(u, b):
            off = pl.multiple_of(u * list_len, 8)
            descs = []
            pos = 0
            for s in sub_sizes:
                descs.append(pltpu.make_async_copy(
                    itab_hbm.at[iidx.at[pl.ds(off + pos, s)]],
                    rows[b].at[pl.ds(pos, s)],
                    sems[b]))
                pos += s
            return descs

        def start_gather(u, b):
            for d in gather_descs(u, b):
                d.start()

        def wait_gather(u, b):
            for d in gather_descs(u, b):
                d.wait()

        def compute(u, rowsb):
            for g in range(ngroups):
                ridx = lax.iota(jnp.int32, _LANES) + (g * _LANES)
                acc = jnp.zeros((_LANES,), jnp.float32)
                for d in range(dim):
                    col = jnp.full((_LANES,), d, jnp.int32)
                    v = plsc.load_gather(rowsb, [ridx, col])
                    acc = acc + v * urows[u, d]
                scores[pl.ds(g * _LANES, _LANES)] = acc

        start_gather(0, 0)

        def body(uu, carry):
            for b in range(2):
                u = uu * 2 + b
                wait_gather(u, b)

                @pl.when(u + 1 < users_per_w)
                def _prefetch():
                    start_gather(u + 1, 1 - b)

                compute(u, rows[b])
                pltpu.sync_copy(
                    scores.at[pl.ds(0, list_len)],
                    out_hbm.at[pl.ds((ubase + u) * list_len, list_len)])
            return carry

        lax.fori_loop(0, users_per_w // 2, body, 0)

    return sc_kernel


def kernel(user_id, item_ids, user_table, item_table):
    batch, list_len = item_ids.shape
    dim = user_table.shape[1]
    sc = _make_sc_kernel(batch, list_len, dim)
    out = sc(user_id.astype(jnp.int32),
             item_ids.reshape(-1).astype(jnp.int32),
             user_table, item_table)
    return out.reshape(batch, list_len)


# speedup vs baseline: 1.0265x; 1.0265x over previous
import functools

import jax
import jax.numpy as jnp
from jax import lax
from jax.experimental import pallas as pl
from jax.experimental.pallas import tpu as pltpu
from jax.experimental.pallas import tpu_sc as plsc

_LANES = 16
_IDX_CHUNK = 128


@functools.lru_cache(maxsize=None)
def _make_sc_kernel(batch, list_len, dim):
    info = plsc.get_sparse_core_info()
    num_workers = info.num_cores * info.num_subcores
    users_per_w = batch // num_workers
    assert batch % num_workers == 0
    ngroups = (list_len + _LANES - 1) // _LANES
    pad_rows = ngroups * _LANES
    sub_sizes = []
    rem = list_len
    while rem > 0:
        s = min(_IDX_CHUNK, rem)
        sub_sizes.append(s)
        rem -= s

    mesh = plsc.VectorSubcoreMesh(core_axis_name="c", subcore_axis_name="s")

    @functools.partial(
        pl.kernel,
        out_type=jax.ShapeDtypeStruct((batch * list_len,), jnp.float32),
        mesh=mesh,
        compiler_params=pltpu.CompilerParams(
            needs_layout_passes=False, use_tc_tiling_on_sc=False),
        scratch_types=[
            pltpu.VMEM((users_per_w,), jnp.int32),
            pltpu.VMEM((users_per_w, dim), jnp.float32),
            pltpu.VMEM((users_per_w * list_len,), jnp.int32),
            pltpu.VMEM((pad_rows, dim), jnp.float32),
            pltpu.VMEM((pad_rows, dim), jnp.float32),
            pltpu.VMEM((pad_rows,), jnp.float32),
            pltpu.SemaphoreType.DMA,
            pltpu.SemaphoreType.DMA,
            pltpu.SemaphoreType.DMA,
        ],
    )
    def sc_kernel(uid_hbm, iid_hbm, utab_hbm, itab_hbm, out_hbm,
                  uidx, urows, iidx, rows0, rows1, scores, sem0, sem1, semu):
        rows = (rows0, rows1)
        sems = (sem0, sem1)
        wid = lax.axis_index("s") * info.num_cores + lax.axis_index("c")
        ubase = wid * users_per_w

        pltpu.sync_copy(uid_hbm.at[pl.ds(ubase, users_per_w)], uidx)
        pltpu.sync_copy(
            iid_hbm.at[pl.ds(ubase * list_len, users_per_w * list_len)], iidx)
        pltpu.async_copy(utab_hbm.at[uidx], urows, semu).wait()

        def gather_descs(u, b):
            off = pl.multiple_of(u * list_len, 8)
            descs = []
            pos = 0
            for s in sub_sizes:
                descs.append(pltpu.make_async_copy(
                    itab_hbm.at[iidx.at[pl.ds(off + pos, s)]],
                    rows[b].at[pl.ds(pos, s)],
                    sems[b]))
                pos += s
            return descs

        def start_gather(u, b):
            for d in gather_descs(u, b):
                d.start()

        def wait_gather(u, b):
            for d in gather_descs(u, b):
                d.wait()

        def compute(u, rowsb):
            uvecs = [urows[u, pl.ds(h * _LANES, _LANES)]
                     for h in range(dim // _LANES)]
            uscal = [uvecs[d // _LANES][d % _LANES] for d in range(dim)]
            for g in range(ngroups):
                ridx = lax.iota(jnp.int32, _LANES) + (g * _LANES)
                acc = jnp.zeros((_LANES,), jnp.float32)
                for d in range(dim):
                    col = jnp.full((_LANES,), d, jnp.int32)
                    v = plsc.load_gather(rowsb, [ridx, col])
                    acc = acc + v * uscal[d]
                scores[pl.ds(g * _LANES, _LANES)] = acc

        start_gather(0, 0)

        def body(uu, carry):
            for b in range(2):
                u = uu * 2 + b
                wait_gather(u, b)

                @pl.when(u + 1 < users_per_w)
                def _prefetch():
                    start_gather(u + 1, 1 - b)

                compute(u, rows[b])
                pltpu.sync_copy(
                    scores.at[pl.ds(0, list_len)],
                    out_hbm.at[pl.ds((ubase + u) * list_len, list_len)])
            return carry

        lax.fori_loop(0, users_per_w // 2, body, 0)

    return sc_kernel


def kernel(user_id, item_ids, user_table, item_table):
    batch, list_len = item_ids.shape
    dim = user_table.shape[1]
    sc = _make_sc_kernel(batch, list_len, dim)
    out = sc(user_id.astype(jnp.int32),
             item_ids.reshape(-1).astype(jnp.int32),
             user_table, item_table)
    return out.reshape(batch, list_len)
